# Initial kernel scaffold; baseline (speedup 1.0000x reference)
import jax
import jax.numpy as jnp
from jax.experimental import pallas as pl


def kernel(x, neighbours):
    raise NotImplementedError("write your pallas kernel here")


# speedup vs baseline: 10.7825x; 10.7825x over previous
import functools

import jax
import jax.numpy as jnp
from jax import lax
from jax.experimental import pallas as pl
from jax.experimental.pallas import tpu as pltpu
from jax.experimental.pallas import tpu_sc as plsc

N_POINTS = 100000
N_NEIGH = 6
D_FEAT = 128

B = 125
G = N_POINTS // B
NW = 32
CPW = G // NW
LANES = 16


def _make_kernel():
    mesh = plsc.VectorSubcoreMesh(
        core_axis_name="c", subcore_axis_name="s",
        num_cores=2, num_subcores=16)

    scratch = (
        [pltpu.VMEM((N_NEIGH, B), jnp.int32)]
        + [pltpu.VMEM((B, D_FEAT), jnp.float32) for _ in range(N_NEIGH)]
        + [pltpu.VMEM((B, D_FEAT), jnp.float32),
           pltpu.SemaphoreType.DMA]
    )

    @functools.partial(
        pl.kernel,
        out_type=jax.ShapeDtypeStruct((N_POINTS, D_FEAT), jnp.float32),
        mesh=mesh,
        scratch_types=scratch,
        compiler_params=pltpu.CompilerParams(use_tc_tiling_on_sc=False),
    )
    def k(x_hbm, nbr_hbm, out_hbm, idx_v, b0, b1, b2, b3, b4, b5, out_v, sem):
        wid = lax.axis_index("s") * 2 + lax.axis_index("c")
        bufs = [b0, b1, b2, b3, b4, b5]
        inv = jnp.float32(1.0 / N_NEIGH)

        def chunk_body(i, carry):
            g = wid * CPW + i
            pltpu.sync_copy(nbr_hbm.at[g], idx_v)
            cps = [pltpu.async_copy(x_hbm.at[idx_v.at[j]], bufs[j], sem)
                   for j in range(N_NEIGH)]
            for cp in cps:
                cp.wait()

            def row_body(b, c2):
                for c in range(D_FEAT // LANES):
                    sl = pl.ds(c * LANES, LANES)
                    s = bufs[0][b, sl]
                    for j in range(1, N_NEIGH):
                        s = s + bufs[j][b, sl]
                    out_v[b, sl] = s * inv
                return c2

            lax.fori_loop(0, B, row_body, 0)
            pltpu.sync_copy(out_v, out_hbm.at[pl.ds(g * B, B)])
            return carry

        lax.fori_loop(0, CPW, chunk_body, 0)

    return k


def kernel(x, neighbours):
    nbr3 = neighbours.reshape(G, B, N_NEIGH).transpose(0, 2, 1)
    return _make_kernel()(x, nbr3)


# speedup vs baseline: 13.1212x; 1.2169x over previous
import functools

import jax
import jax.numpy as jnp
from jax import lax
from jax.experimental import pallas as pl
from jax.experimental.pallas import tpu as pltpu
from jax.experimental.pallas import tpu_sc as plsc

N_POINTS = 100000
N_NEIGH = 6
D_FEAT = 128

B = 125
G = N_POINTS // B
NW = 32
CPW = G // NW
LANES = 16


def _make_kernel():
    mesh = plsc.VectorSubcoreMesh(
        core_axis_name="c", subcore_axis_name="s",
        num_cores=2, num_subcores=16)

    scratch = (
        [pltpu.VMEM((N_NEIGH, B), jnp.int32),
         pltpu.VMEM((B, D_FEAT), jnp.float32),
         pltpu.VMEM((B, D_FEAT), jnp.float32),
         pltpu.VMEM((B, D_FEAT), jnp.float32),
         pltpu.SemaphoreType.DMA]
    )

    @functools.partial(
        pl.kernel,
        out_type=jax.ShapeDtypeStruct((N_POINTS, D_FEAT), jnp.float32),
        mesh=mesh,
        scratch_types=scratch,
        compiler_params=pltpu.CompilerParams(use_tc_tiling_on_sc=False),
    )
    def k(x_hbm, nbr_hbm, out_hbm, idx_v, buf_a, buf_b, out_v, sem):
        wid = lax.axis_index("s") * 2 + lax.axis_index("c")
        inv = jnp.float32(1.0 / N_NEIGH)
        zeros = jnp.zeros((LANES,), jnp.float32)

        def chunk_body(i, carry):
            g = wid * CPW + i

            def zero_body(b, c2):
                for c in range(D_FEAT // LANES):
                    buf_b[b, pl.ds(c * LANES, LANES)] = zeros
                return c2
            lax.fori_loop(0, B, zero_body, 0)

            pltpu.sync_copy(nbr_hbm.at[g], idx_v)
            cps = [pltpu.async_copy(x_hbm.at[idx_v.at[0]], buf_a, sem)]
            for j in range(1, N_NEIGH):
                cps.append(pltpu.async_copy(
                    x_hbm.at[idx_v.at[j]], buf_b, sem, add=True))
            for cp in cps:
                cp.wait()

            def row_body(b, c2):
                for c in range(D_FEAT // LANES):
                    sl = pl.ds(c * LANES, LANES)
                    out_v[b, sl] = (buf_a[b, sl] + buf_b[b, sl]) * inv
                return c2

            lax.fori_loop(0, B, row_body, 0)
            pltpu.sync_copy(out_v, out_hbm.at[pl.ds(g * B, B)])
            return carry

        lax.fori_loop(0, CPW, chunk_body, 0)

    return k


def kernel(x, neighbours):
    nbr3 = neighbours.reshape(G, B, N_NEIGH).transpose(0, 2, 1)
    return _make_kernel()(x, nbr3)


# speedup vs baseline: 17.6682x; 1.3465x over previous
import functools

import jax
import jax.numpy as jnp
from jax import lax
from jax.experimental import pallas as pl
from jax.experimental.pallas import tpu as pltpu
from jax.experimental.pallas import tpu_sc as plsc

N_POINTS = 100000
N_NEIGH = 6
D_FEAT = 128

B = 125
G = N_POINTS // B
NW = 32
CPW = G // NW
LANES = 16


def _make_kernel():
    mesh = plsc.VectorSubcoreMesh(
        core_axis_name="c", subcore_axis_name="s",
        num_cores=2, num_subcores=16)

    scratch = [
        pltpu.VMEM((CPW, N_NEIGH, B), jnp.int32),
        pltpu.VMEM((B, D_FEAT), jnp.float32),
        pltpu.VMEM((B, D_FEAT), jnp.float32),
        pltpu.VMEM((B, D_FEAT), jnp.float32),
        pltpu.VMEM((B, D_FEAT), jnp.float32),
        pltpu.VMEM((B, D_FEAT), jnp.float32),
        pltpu.VMEM((B, D_FEAT), jnp.float32),
        pltpu.SemaphoreType.DMA,
        pltpu.SemaphoreType.DMA,
        pltpu.SemaphoreType.DMA,
        pltpu.SemaphoreType.DMA,
    ]

    @functools.partial(
        pl.kernel,
        out_type=jax.ShapeDtypeStruct((N_POINTS, D_FEAT), jnp.float32),
        mesh=mesh,
        scratch_types=scratch,
        compiler_params=pltpu.CompilerParams(use_tc_tiling_on_sc=False),
    )
    def k(x_hbm, nbr_hbm, out_hbm, idx_all,
          a0, a1, b0, b1, ov0, ov1, gs0, gs1, os0, os1):
        wid = lax.axis_index("s") * 2 + lax.axis_index("c")
        base_g = wid * CPW
        a = [a0, a1]
        bb = [b0, b1]
        ov = [ov0, ov1]
        gs = [gs0, gs1]
        osem = [os0, os1]
        inv = jnp.float32(1.0 / N_NEIGH)
        zeros = jnp.zeros((LANES,), jnp.float32)

        pltpu.sync_copy(nbr_hbm.at[pl.ds(base_g, CPW)], idx_all)

        def zero_buf(buf):
            def zb(r, c2):
                for c in range(D_FEAT // LANES):
                    buf[r, pl.ds(c * LANES, LANES)] = zeros
                return c2
            lax.fori_loop(0, B, zb, 0)

        zero_buf(b0)
        zero_buf(b1)

        def fire6(i):
            s = i % 2
            cps = [pltpu.async_copy(x_hbm.at[idx_all.at[i, 0]], a[s], gs[s])]
            for j in range(1, N_NEIGH):
                cps.append(pltpu.async_copy(
                    x_hbm.at[idx_all.at[i, j]], bb[s], gs[s], add=True))
            return cps

        def compute(i):
            s = i % 2
            av, bv, rv = a[s], bb[s], ov[s]

            def row(r, c2):
                for c in range(D_FEAT // LANES):
                    sl = pl.ds(c * LANES, LANES)
                    rv[r, sl] = (av[r, sl] + bv[r, sl]) * inv
                    bv[r, sl] = zeros
                return c2
            lax.fori_loop(0, B, row, 0)

        handles = fire6(0)
        out_handles = [None] * CPW
        for i in range(CPW):
            nxt_handles = fire6(i + 1) if i + 1 < CPW else None
            for cp in handles:
                cp.wait()
            handles = nxt_handles
            if i >= 2:
                out_handles[i - 2].wait()
            compute(i)
            s = i % 2
            out_handles[i] = pltpu.async_copy(
                ov[s], out_hbm.at[pl.ds((base_g + i) * B, B)], osem[s])
        out_handles[CPW - 2].wait()
        out_handles[CPW - 1].wait()

    return k


def kernel(x, neighbours):
    nbr3 = neighbours.reshape(G, B, N_NEIGH).transpose(0, 2, 1)
    return _make_kernel()(x, nbr3)


# speedup vs baseline: 17.9248x; 1.0145x over previous
import functools

import jax
import jax.numpy as jnp
from jax import lax
from jax.experimental import pallas as pl
from jax.experimental.pallas import tpu as pltpu
from jax.experimental.pallas import tpu_sc as plsc

N_POINTS = 100000
N_NEIGH = 6
D_FEAT = 128

B = 125
G = N_POINTS // B
NW = 32
CPW = G // NW
LANES = 16
NS = 3


def _make_kernel():
    mesh = plsc.VectorSubcoreMesh(
        core_axis_name="c", subcore_axis_name="s",
        num_cores=2, num_subcores=16)

    scratch = (
        [pltpu.VMEM((CPW, N_NEIGH, B), jnp.int32)]
        + [pltpu.VMEM((B, D_FEAT), jnp.float32) for _ in range(2 * NS)]
        + [pltpu.SemaphoreType.DMA for _ in range(2 * NS)]
    )

    @functools.partial(
        pl.kernel,
        out_type=jax.ShapeDtypeStruct((N_POINTS, D_FEAT), jnp.float32),
        mesh=mesh,
        scratch_types=scratch,
        compiler_params=pltpu.CompilerParams(use_tc_tiling_on_sc=False),
    )
    def k(x_hbm, nbr_hbm, out_hbm, idx_all,
          a0, a1, a2, b0, b1, b2, gs0, gs1, gs2, os0, os1, os2):
        wid = lax.axis_index("s") * 2 + lax.axis_index("c")
        base_g = wid * CPW
        a = [a0, a1, a2]
        bb = [b0, b1, b2]
        gs = [gs0, gs1, gs2]
        osem = [os0, os1, os2]
        inv = jnp.float32(1.0 / N_NEIGH)
        zeros = jnp.zeros((LANES,), jnp.float32)

        pltpu.sync_copy(nbr_hbm.at[pl.ds(base_g, CPW)], idx_all)

        def zero_buf(buf):
            def zb(r, c2):
                for c in range(D_FEAT // LANES):
                    buf[r, pl.ds(c * LANES, LANES)] = zeros
                return c2
            lax.fori_loop(0, B, zb, 0)

        for buf in bb:
            zero_buf(buf)

        def fire6(i):
            s = i % NS
            cps = [pltpu.async_copy(x_hbm.at[idx_all.at[i, 0]], a[s], gs[s])]
            for j in range(1, N_NEIGH):
                cps.append(pltpu.async_copy(
                    x_hbm.at[idx_all.at[i, j]], bb[s], gs[s], add=True))
            return cps

        def compute(i):
            s = i % NS
            av, bv = a[s], bb[s]

            def row(r, c2):
                for c in range(D_FEAT // LANES):
                    sl = pl.ds(c * LANES, LANES)
                    av[r, sl] = (av[r, sl] + bv[r, sl]) * inv
                    bv[r, sl] = zeros
                return c2
            lax.fori_loop(0, B, row, 0)

        handles = [None] * CPW
        out_handles = [None] * CPW
        handles[0] = fire6(0)
        handles[1] = fire6(1)
        for i in range(CPW):
            if i + 2 < CPW:
                if i - 1 >= 0:
                    out_handles[i - 1].wait()
                handles[i + 2] = fire6(i + 2)
            for cp in handles[i]:
                cp.wait()
            compute(i)
            s = i % NS
            out_handles[i] = pltpu.async_copy(
                a[s], out_hbm.at[pl.ds((base_g + i) * B, B)], osem[s])
        for i in range(CPW - 3, CPW):
            out_handles[i].wait()

    return k


def kernel(x, neighbours):
    nbr3 = neighbours.reshape(G, B, N_NEIGH).transpose(0, 2, 1)
    return _make_kernel()(x, nbr3)
